# baseline (speedup 1.0000x reference)
import functools

import jax
import jax.numpy as jnp
from jax import lax
from jax.experimental import pallas as pl
from jax.experimental.pallas import tpu as pltpu
from jax.experimental.pallas import tpu_sc as plsc


@functools.lru_cache(maxsize=None)
def _make_gather(V, D, B, n_chunks=8):
    info = plsc.get_sparse_core_info()
    NC, NS = info.num_cores, info.num_subcores
    NW = NC * NS
    assert B % (NW * n_chunks) == 0
    b_per_w = B // NW
    chunk = b_per_w // n_chunks
    mesh = plsc.VectorSubcoreMesh(core_axis_name="c", subcore_axis_name="s")

    @functools.partial(
        pl.kernel,
        mesh=mesh,
        out_type=jax.ShapeDtypeStruct((B, D), jnp.float32),
        scratch_types=[
            pltpu.VMEM((b_per_w,), jnp.int32),
            *[pltpu.VMEM((chunk, D), jnp.float32) for _ in range(n_chunks)],
            *[pltpu.SemaphoreType.DMA for _ in range(n_chunks)],
            pltpu.SemaphoreType.DMA,
        ],
    )
    def k(idx_hbm, table_hbm, out_hbm, idx_v, *rest):
        bufs = rest[:n_chunks]
        gsems = rest[n_chunks : 2 * n_chunks]
        wsem = rest[2 * n_chunks]
        wid = lax.axis_index("s") * NC + lax.axis_index("c")
        base = wid * b_per_w
        pltpu.sync_copy(idx_hbm.at[pl.ds(base, b_per_w)], idx_v)

        def g_start(c):
            return pltpu.async_copy(
                table_hbm.at[idx_v.at[pl.ds(c * chunk, chunk)]], bufs[c], gsems[c]
            )

        gathers = [None] * n_chunks
        gathers[0] = g_start(0)
        writes = []
        for c in range(n_chunks):
            if c + 1 < n_chunks:
                gathers[c + 1] = g_start(c + 1)
            gathers[c].wait()
            writes.append(
                pltpu.async_copy(bufs[c], out_hbm.at[pl.ds(base + c * chunk, chunk)], wsem)
            )
        for w in writes:
            w.wait()

    return k


def kernel(node_ids, memory):
    V, D = memory.shape
    (B,) = node_ids.shape
    f = _make_gather(V, D, B)
    return f(node_ids.astype(jnp.int32), memory)


# speedup vs baseline: 1.0706x; 1.0706x over previous
import functools

import jax
import jax.numpy as jnp
from jax import lax
from jax.experimental import pallas as pl
from jax.experimental.pallas import tpu as pltpu
from jax.experimental.pallas import tpu_sc as plsc


@functools.lru_cache(maxsize=None)
def _make_gather(V, D, B):
    info = plsc.get_sparse_core_info()
    NC, NS = info.num_cores, info.num_subcores
    NW = NC * NS
    assert B % (2 * NW) == 0
    b_per_w = B // NW
    half = b_per_w // 2
    mesh = plsc.VectorSubcoreMesh(core_axis_name="c", subcore_axis_name="s")

    @functools.partial(
        pl.kernel,
        mesh=mesh,
        out_type=jax.ShapeDtypeStruct((B, D), jnp.float32),
        scratch_types=[
            pltpu.VMEM((b_per_w,), jnp.int32),
            pltpu.VMEM((half, D), jnp.float32),
            pltpu.VMEM((half, D), jnp.float32),
            pltpu.SemaphoreType.DMA,
            pltpu.SemaphoreType.DMA,
            pltpu.SemaphoreType.DMA,
            pltpu.SemaphoreType.DMA,
        ],
    )
    def k(idx_hbm, table_hbm, out_hbm, idx_v, buf0, buf1, isem, g0sem, g1sem, wsem):
        wid = lax.axis_index("s") * NC + lax.axis_index("c")
        base = wid * b_per_w
        pltpu.sync_copy(idx_hbm.at[pl.ds(base, half)], idx_v.at[pl.ds(0, half)])
        g0 = pltpu.async_copy(table_hbm.at[idx_v.at[pl.ds(0, half)]], buf0, g0sem)
        pltpu.async_copy(
            idx_hbm.at[pl.ds(base + half, half)], idx_v.at[pl.ds(half, half)], isem
        ).wait()
        g1 = pltpu.async_copy(table_hbm.at[idx_v.at[pl.ds(half, half)]], buf1, g1sem)
        g0.wait()
        w0 = pltpu.async_copy(buf0, out_hbm.at[pl.ds(base, half)], wsem)
        g1.wait()
        w1 = pltpu.async_copy(buf1, out_hbm.at[pl.ds(base + half, half)], wsem)
        w0.wait()
        w1.wait()

    return k


def kernel(node_ids, memory):
    V, D = memory.shape
    (B,) = node_ids.shape
    f = _make_gather(V, D, B)
    return f(node_ids.astype(jnp.int32), memory)
